# Initial kernel scaffold; baseline (speedup 1.0000x reference)
import functools

import jax
import jax.numpy as jnp
from jax import lax
from jax.experimental import pallas as pl
from jax.experimental.pallas import tpu as pltpu
from jax.experimental.pallas import tpu_sc as plsc

N = 10000
D = 128
E = 320000
NC, NS = 2, 16
NW = NC * NS
CHUNK = 125
CPW = E // (NW * CHUNK)
RPT = N // NS
ZBLK = 125


def _sc_aggregate(with_cnt):
    mesh = plsc.VectorSubcoreMesh(
        core_axis_name="c", subcore_axis_name="s", num_cores=NC, num_subcores=NS
    )
    out_type = [jax.ShapeDtypeStruct((NC, N, D), jnp.float32)]
    scratch = [
        pltpu.VMEM((CPW, CHUNK), jnp.int32),
        pltpu.VMEM((CPW, CHUNK), jnp.int32),
        pltpu.VMEM((CHUNK, D), jnp.float32),
        pltpu.VMEM_SHARED((N, D), jnp.float32),
        pltpu.SemaphoreType.DMA,
    ]
    if with_cnt:
        out_type.append(jax.ShapeDtypeStruct((NC, N, 16), jnp.float32))
        scratch += [
            pltpu.VMEM((CHUNK, 16), jnp.float32),
            pltpu.VMEM_SHARED((N, 16), jnp.float32),
        ]

    def body(feat_hbm, src_hbm, dst_hbm, sum_hbm, *rest):
        if with_cnt:
            cnt_hbm, src_v, dst_v, rows_v, acc_sh, sem, ones_v, cnt_sh = rest
        else:
            src_v, dst_v, rows_v, acc_sh, sem = rest
        c = lax.axis_index("c")
        s = lax.axis_index("s")
        wid = c * NS + s
        row0 = s * RPT

        @pl.loop(0, CHUNK)
        def _(i):
            @pl.loop(0, D // 16)
            def _(j):
                rows_v[i, pl.ds(j * 16, 16)] = jnp.zeros((16,), jnp.float32)

        if with_cnt:
            @pl.loop(0, CHUNK)
            def _(i):
                ones_v[i, pl.ds(0, 16)] = jnp.zeros((16,), jnp.float32)

        @pl.loop(0, RPT // ZBLK)
        def _(z):
            r = row0 + z * ZBLK
            pltpu.sync_copy(rows_v, acc_sh.at[pl.ds(r, ZBLK)])
            if with_cnt:
                pltpu.sync_copy(ones_v, cnt_sh.at[pl.ds(r, ZBLK)])

        if with_cnt:
            @pl.loop(0, CHUNK)
            def _(i):
                ones_v[i, pl.ds(0, 16)] = jnp.ones((16,), jnp.float32)

        pltpu.sync_copy(src_hbm.at[wid], src_v)
        pltpu.sync_copy(dst_hbm.at[wid], dst_v)
        plsc.subcore_barrier()

        @pl.loop(0, CPW)
        def _(j):
            pltpu.sync_copy(feat_hbm.at[src_v.at[j]], rows_v)
            pltpu.sync_copy(rows_v, acc_sh.at[dst_v.at[j]], add=True)
            if with_cnt:
                pltpu.sync_copy(ones_v, cnt_sh.at[dst_v.at[j]], add=True)

        plsc.subcore_barrier()

        pltpu.sync_copy(acc_sh.at[pl.ds(row0, RPT)], sum_hbm.at[c, pl.ds(row0, RPT)])
        if with_cnt:
            pltpu.sync_copy(cnt_sh.at[pl.ds(row0, RPT)], cnt_hbm.at[c, pl.ds(row0, RPT)])

    return pl.kernel(body, out_type=tuple(out_type) if with_cnt else out_type[0],
                     mesh=mesh, scratch_types=scratch)


_sc_aggregate_cnt = _sc_aggregate(True)
_sc_aggregate_nocnt = _sc_aggregate(False)

BLK = 2000


def _tc_layer(sum_parts, cnt_parts, feats, Wl, bl, Wr, relu):
    def body(sum_ref, cnt_ref, x_ref, wl_ref, bl_ref, wr_ref, o_ref):
        ssum = sum_ref[0] + sum_ref[1]
        cnt = cnt_ref[0, :, 0:1] + cnt_ref[1, :, 0:1]
        mean = ssum / jnp.maximum(cnt, 1.0)
        acc = lax.dot_general(mean, wl_ref[...], (((1,), (1,)), ((), ())),
                              preferred_element_type=jnp.float32)
        acc = acc + lax.dot_general(x_ref[...], wr_ref[...],
                                    (((1,), (1,)), ((), ())),
                                    preferred_element_type=jnp.float32)
        acc = acc + bl_ref[...]
        if relu:
            acc = jnp.maximum(acc, 0.0)
        o_ref[...] = acc

    return pl.pallas_call(
        body,
        grid=(N // BLK,),
        in_specs=[
            pl.BlockSpec((2, BLK, D), lambda i: (0, i, 0)),
            pl.BlockSpec((2, BLK, 16), lambda i: (0, i, 0)),
            pl.BlockSpec((BLK, D), lambda i: (i, 0)),
            pl.BlockSpec((D, D), lambda i: (0, 0)),
            pl.BlockSpec((1, D), lambda i: (0, 0)),
            pl.BlockSpec((D, D), lambda i: (0, 0)),
        ],
        out_specs=pl.BlockSpec((BLK, D), lambda i: (i, 0)),
        out_shape=jax.ShapeDtypeStruct((N, D), jnp.float32),
    )(sum_parts, cnt_parts, feats, Wl, bl.reshape(1, D), Wr)


def kernel(x, edge_index, W1l, b1, W1r, W2l, b2, W2r):
    src = edge_index[0].astype(jnp.int32).reshape(NW, CPW, CHUNK)
    dst = edge_index[1].astype(jnp.int32).reshape(NW, CPW, CHUNK)
    sum1, cnt = _sc_aggregate_cnt(x, src, dst)
    h = _tc_layer(sum1, cnt, x, W1l, b1, W1r, relu=True)
    sum2 = _sc_aggregate_nocnt(h, src, dst)
    out = _tc_layer(sum2, cnt, h, W2l, b2, W2r, relu=False)
    return out


# speedup vs baseline: 3.4942x; 3.4942x over previous
import dataclasses
import functools

import jax
import jax.numpy as jnp
from jax import lax
from jax.experimental import pallas as pl
from jax.experimental.pallas import tpu as pltpu
from jax.experimental.pallas import tpu_sc as plsc

N = 10000
NP = 10240
D = 128
E = 320000
NC, NS = 2, 16
NW = NC * NS
CHUNK = 128
CPW = 80
IB = 16
EP = NW * CPW * CHUNK
RPT = NP // NS


@functools.lru_cache(maxsize=None)
def _sc_aggregate(with_cnt):
    mesh = plsc.VectorSubcoreMesh(
        core_axis_name="c", subcore_axis_name="s", num_cores=NC, num_subcores=NS
    )
    out_type = [jax.ShapeDtypeStruct((NC, NP, D), jnp.float32)]
    scratch = [
        pltpu.VMEM((IB, CHUNK), jnp.int32),
        pltpu.VMEM((IB, CHUNK), jnp.int32),
        pltpu.VMEM((CHUNK, D), jnp.float32),
        pltpu.VMEM_SHARED((NP, D), jnp.float32),
        pltpu.SemaphoreType.DMA,
    ]
    if with_cnt:
        out_type.append(jax.ShapeDtypeStruct((NW, NP), jnp.float32))
        scratch.append(pltpu.VMEM((NP,), jnp.float32))

    def body(feat_hbm, src_hbm, dst_hbm, sum_hbm, *rest):
        if with_cnt:
            cnt_hbm, src_v, dst_v, rows_v, acc_sh, sem, hist_v = rest
        else:
            src_v, dst_v, rows_v, acc_sh, sem = rest
        c = lax.axis_index("c")
        s = lax.axis_index("s")
        wid = c * NS + s
        row0 = s * RPT

        @pl.loop(0, CHUNK)
        def _(i):
            @pl.loop(0, D // 16)
            def _(j):
                rows_v[i, pl.ds(j * 16, 16)] = jnp.zeros((16,), jnp.float32)

        if with_cnt:
            @pl.loop(0, NP // 16)
            def _(i):
                hist_v[pl.ds(i * 16, 16)] = jnp.zeros((16,), jnp.float32)

        @pl.loop(0, RPT // CHUNK)
        def _(z):
            pltpu.sync_copy(rows_v, acc_sh.at[pl.ds(row0 + z * CHUNK, CHUNK)])

        plsc.subcore_barrier()

        ones16 = jnp.ones((16,), jnp.float32)

        @pl.loop(0, CPW // IB)
        def _(b):
            pltpu.sync_copy(src_hbm.at[wid, pl.ds(b * IB, IB)], src_v)
            pltpu.sync_copy(dst_hbm.at[wid, pl.ds(b * IB, IB)], dst_v)
            for j in range(IB):
                pltpu.sync_copy(feat_hbm.at[src_v.at[j]], rows_v)
                pltpu.sync_copy(rows_v, acc_sh.at[dst_v.at[j]], add=True)
                if with_cnt:
                    @pl.loop(0, CHUNK // 16)
                    def _(k):
                        idx = dst_v[j, pl.ds(k * 16, 16)]
                        plsc.addupdate_scatter(hist_v, [idx], ones16)

        plsc.subcore_barrier()

        pltpu.sync_copy(acc_sh.at[pl.ds(row0, RPT)], sum_hbm.at[c, pl.ds(row0, RPT)])
        if with_cnt:
            pltpu.sync_copy(hist_v, cnt_hbm.at[wid])

    cp = pltpu.CompilerParams()
    if "needs_layout_passes" in pltpu.CompilerParams.__dataclass_fields__:
        cp = dataclasses.replace(cp, needs_layout_passes=False)
    return pl.kernel(body, out_type=tuple(out_type) if with_cnt else out_type[0],
                     mesh=mesh, scratch_types=scratch, compiler_params=cp)


BLK = 2048


def _tc_layer(sum_parts, inv_cnt, feats, Wl, bl, Wr, relu):
    def body(sum_ref, inv_ref, x_ref, wl_ref, bl_ref, wr_ref, o_ref):
        mean = (sum_ref[0] + sum_ref[1]) * inv_ref[...]
        acc = lax.dot_general(mean, wl_ref[...], (((1,), (1,)), ((), ())),
                              preferred_element_type=jnp.float32)
        acc = acc + lax.dot_general(x_ref[...], wr_ref[...],
                                    (((1,), (1,)), ((), ())),
                                    preferred_element_type=jnp.float32)
        acc = acc + bl_ref[...]
        if relu:
            acc = jnp.maximum(acc, 0.0)
        o_ref[...] = acc

    return pl.pallas_call(
        body,
        grid=(NP // BLK,),
        in_specs=[
            pl.BlockSpec((2, BLK, D), lambda i: (0, i, 0)),
            pl.BlockSpec((BLK, 1), lambda i: (i, 0)),
            pl.BlockSpec((BLK, D), lambda i: (i, 0)),
            pl.BlockSpec((D, D), lambda i: (0, 0)),
            pl.BlockSpec((1, D), lambda i: (0, 0)),
            pl.BlockSpec((D, D), lambda i: (0, 0)),
        ],
        out_specs=pl.BlockSpec((BLK, D), lambda i: (i, 0)),
        out_shape=jax.ShapeDtypeStruct((NP, D), jnp.float32),
    )(sum_parts, inv_cnt, feats, Wl, bl.reshape(1, D), Wr)


def kernel(x, edge_index, W1l, b1, W1r, W2l, b2, W2r):
    pad = EP - E
    src = jnp.concatenate(
        [edge_index[0].astype(jnp.int32), jnp.zeros((pad,), jnp.int32)]
    ).reshape(NW, CPW, CHUNK)
    dst = jnp.concatenate(
        [edge_index[1].astype(jnp.int32),
         N + (jnp.arange(pad, dtype=jnp.int32) % (NP - N))]
    ).reshape(NW, CPW, CHUNK)
    x_pad = jnp.pad(x, ((0, NP - N), (0, 0)))
    sum1, cnt_parts = _sc_aggregate(True)(x_pad, src, dst)
    inv_cnt = (1.0 / jnp.maximum(cnt_parts.sum(axis=0), 1.0)).reshape(NP, 1)
    h = _tc_layer(sum1, inv_cnt, x_pad, W1l, b1, W1r, relu=True)
    sum2 = _sc_aggregate(False)(h, src, dst)
    out = _tc_layer(sum2, inv_cnt, h, W2l, b2, W2r, relu=False)
    return out[:N]


# speedup vs baseline: 3.7713x; 1.0793x over previous
import dataclasses
import functools

import jax
import jax.numpy as jnp
from jax import lax
from jax.experimental import pallas as pl
from jax.experimental.pallas import tpu as pltpu
from jax.experimental.pallas import tpu_sc as plsc

N = 10000
NP = 10240
D = 128
E = 320000
NC, NS = 2, 16
NW = NC * NS
CHUNK = 128
CPW = 80
EP = NW * CPW * CHUNK
RPT = NP // NS


@functools.lru_cache(maxsize=None)
def _sc_aggregate(with_cnt):
    IB = 4 if with_cnt else 16
    mesh = plsc.VectorSubcoreMesh(
        core_axis_name="c", subcore_axis_name="s", num_cores=NC, num_subcores=NS
    )
    out_type = [jax.ShapeDtypeStruct((NC, NP, D), jnp.float32)]
    scratch = [
        pltpu.VMEM((IB, CHUNK), jnp.int32),
        pltpu.VMEM((IB, CHUNK), jnp.int32),
        pltpu.VMEM((CHUNK, D), jnp.float32),
        pltpu.VMEM((CHUNK, D), jnp.float32),
        pltpu.VMEM_SHARED((NP, D), jnp.float32),
        pltpu.SemaphoreType.DMA,
        pltpu.SemaphoreType.DMA,
    ]
    if with_cnt:
        out_type.append(jax.ShapeDtypeStruct((NW, NP), jnp.float32))
        scratch.append(pltpu.VMEM((NP,), jnp.float32))

    def body(feat_hbm, src_hbm, dst_hbm, *rest):
        if with_cnt:
            (sum_hbm, cnt_hbm, src_v, dst_v, rows0, rows1, acc_sh,
             sg0, sg1, hist_v) = rest
        else:
            sum_hbm, src_v, dst_v, rows0, rows1, acc_sh, sg0, sg1 = rest
        rows = (rows0, rows1)
        sem_g = (sg0, sg1)
        c = lax.axis_index("c")
        s = lax.axis_index("s")
        wid = c * NS + s
        row0 = s * RPT

        @pl.loop(0, CHUNK)
        def _(i):
            @pl.loop(0, D // 16)
            def _(j):
                rows0[i, pl.ds(j * 16, 16)] = jnp.zeros((16,), jnp.float32)

        if with_cnt:
            @pl.loop(0, NP // 16)
            def _(i):
                hist_v[pl.ds(i * 16, 16)] = jnp.zeros((16,), jnp.float32)

        @pl.loop(0, RPT // CHUNK)
        def _(z):
            pltpu.sync_copy(rows0, acc_sh.at[pl.ds(row0 + z * CHUNK, CHUNK)])

        plsc.subcore_barrier()

        ones16 = jnp.ones((16,), jnp.float32)

        @pl.loop(0, CPW // IB)
        def _(b):
            pltpu.sync_copy(src_hbm.at[wid, pl.ds(b * IB, IB)], src_v)
            pltpu.sync_copy(dst_hbm.at[wid, pl.ds(b * IB, IB)], dst_v)
            dg = [None] * IB
            dg[0] = pltpu.async_copy(feat_hbm.at[src_v.at[0]], rows[0], sem_g[0])
            for j in range(IB):
                p = j % 2
                if j + 1 < IB:
                    dg[j + 1] = pltpu.async_copy(
                        feat_hbm.at[src_v.at[j + 1]], rows[1 - p], sem_g[1 - p])
                dg[j].wait()
                pltpu.sync_copy(rows[p], acc_sh.at[dst_v.at[j]], add=True)
                if with_cnt:
                    @pl.loop(0, CHUNK // 16)
                    def _(k):
                        idx = dst_v[j, pl.ds(k * 16, 16)]
                        plsc.addupdate_scatter(hist_v, [idx], ones16)

        plsc.subcore_barrier()

        pltpu.sync_copy(acc_sh.at[pl.ds(row0, RPT)], sum_hbm.at[c, pl.ds(row0, RPT)])
        if with_cnt:
            pltpu.sync_copy(hist_v, cnt_hbm.at[wid])

    cp = pltpu.CompilerParams()
    if "needs_layout_passes" in pltpu.CompilerParams.__dataclass_fields__:
        cp = dataclasses.replace(cp, needs_layout_passes=False)
    return pl.kernel(body, out_type=tuple(out_type) if with_cnt else out_type[0],
                     mesh=mesh, scratch_types=scratch, compiler_params=cp)


BLK = 2048


def _tc_layer(sum_parts, inv_cnt, feats, Wl, bl, Wr, relu):
    def body(sum_ref, inv_ref, x_ref, wl_ref, bl_ref, wr_ref, o_ref):
        mean = (sum_ref[0] + sum_ref[1]) * inv_ref[...]
        acc = lax.dot_general(mean, wl_ref[...], (((1,), (1,)), ((), ())),
                              preferred_element_type=jnp.float32)
        acc = acc + lax.dot_general(x_ref[...], wr_ref[...],
                                    (((1,), (1,)), ((), ())),
                                    preferred_element_type=jnp.float32)
        acc = acc + bl_ref[...]
        if relu:
            acc = jnp.maximum(acc, 0.0)
        o_ref[...] = acc

    return pl.pallas_call(
        body,
        grid=(NP // BLK,),
        in_specs=[
            pl.BlockSpec((2, BLK, D), lambda i: (0, i, 0)),
            pl.BlockSpec((BLK, 1), lambda i: (i, 0)),
            pl.BlockSpec((BLK, D), lambda i: (i, 0)),
            pl.BlockSpec((D, D), lambda i: (0, 0)),
            pl.BlockSpec((1, D), lambda i: (0, 0)),
            pl.BlockSpec((D, D), lambda i: (0, 0)),
        ],
        out_specs=pl.BlockSpec((BLK, D), lambda i: (i, 0)),
        out_shape=jax.ShapeDtypeStruct((NP, D), jnp.float32),
    )(sum_parts, inv_cnt, feats, Wl, bl.reshape(1, D), Wr)


def kernel(x, edge_index, W1l, b1, W1r, W2l, b2, W2r):
    pad = EP - E
    src = jnp.concatenate(
        [edge_index[0].astype(jnp.int32), jnp.zeros((pad,), jnp.int32)]
    ).reshape(NW, CPW, CHUNK)
    dst_flat = jnp.concatenate(
        [edge_index[1].astype(jnp.int32),
         N + (jnp.arange(pad, dtype=jnp.int32) % (NP - N))]
    )
    dst = dst_flat.reshape(NW, CPW, CHUNK)
    x_pad = jnp.pad(x, ((0, NP - N), (0, 0)))
    sum1, cnt_parts = _sc_aggregate(True)(x_pad, src, dst)
    inv_cnt = (1.0 / jnp.maximum(cnt_parts.sum(axis=0), 1.0)).reshape(NP, 1)
    h = _tc_layer(sum1, inv_cnt, x_pad, W1l, b1, W1r, relu=True)
    sum2 = _sc_aggregate(False)(h, src, dst)
    out = _tc_layer(sum2, inv_cnt, h, W2l, b2, W2r, relu=False)
    return out[:N]


# speedup vs baseline: 4.5674x; 1.2111x over previous
import dataclasses
import functools

import jax
import jax.numpy as jnp
from jax import lax
from jax.experimental import pallas as pl
from jax.experimental.pallas import tpu as pltpu
from jax.experimental.pallas import tpu_sc as plsc

N = 10000
NP = 10240
D = 128
E = 320000
NC, NS = 2, 16
NW = NC * NS
EP = 327680
RPT = NP // NS
IB = 8
FAST_CORE = 0


@functools.lru_cache(maxsize=None)
def _sc_aggregate(with_cnt):
    CHUNK = 64
    NCH = EP // CHUNK
    CPT0 = (NCH * 9 // 10) // NS
    CPT0 -= CPT0 % IB
    CPT1 = (NCH - NS * CPT0) // NS
    assert NS * (CPT0 + CPT1) == NCH and CPT1 % IB == 0
    NB0, NB1 = CPT0 // IB, CPT1 // IB

    mesh = plsc.VectorSubcoreMesh(
        core_axis_name="c", subcore_axis_name="s", num_cores=NC, num_subcores=NS
    )
    out_type = [jax.ShapeDtypeStruct((NC, NP, D), jnp.float32)]
    scratch = [
        pltpu.VMEM((IB, CHUNK), jnp.int32),
        pltpu.VMEM((IB, CHUNK), jnp.int32),
        pltpu.VMEM((CHUNK, D), jnp.float32),
        pltpu.VMEM((CHUNK, D), jnp.float32),
        pltpu.VMEM_SHARED((NP, D), jnp.float32),
        pltpu.SemaphoreType.DMA,
        pltpu.SemaphoreType.DMA,
    ]
    if with_cnt:
        out_type.append(jax.ShapeDtypeStruct((NW, NP), jnp.float32))
        scratch.append(pltpu.VMEM((NP,), jnp.float32))

    def body(feat_hbm, src_hbm, dst_hbm, *rest):
        if with_cnt:
            (sum_hbm, cnt_hbm, src_v, dst_v, rows0, rows1, acc_sh,
             sg0, sg1, hist_v) = rest
        else:
            sum_hbm, src_v, dst_v, rows0, rows1, acc_sh, sg0, sg1 = rest
        rows = (rows0, rows1)
        sem_g = (sg0, sg1)
        c = lax.axis_index("c")
        s = lax.axis_index("s")
        wid = c * NS + s
        row0 = s * RPT
        big = (c == FAST_CORE).astype(jnp.int32)
        nblocks = NB1 + big * (NB0 - NB1)
        base = (1 - big) * (NS * CPT0) + s * (CPT1 + big * (CPT0 - CPT1))

        @pl.loop(0, CHUNK)
        def _(i):
            @pl.loop(0, D // 16)
            def _(j):
                rows0[i, pl.ds(j * 16, 16)] = jnp.zeros((16,), jnp.float32)

        if with_cnt:
            @pl.loop(0, NP // 16)
            def _(i):
                hist_v[pl.ds(i * 16, 16)] = jnp.zeros((16,), jnp.float32)

        @pl.loop(0, RPT // CHUNK)
        def _(z):
            pltpu.sync_copy(rows0, acc_sh.at[pl.ds(row0 + z * CHUNK, CHUNK)])

        plsc.subcore_barrier()

        ones16 = jnp.ones((16,), jnp.float32)

        @pl.loop(0, nblocks)
        def _(b):
            blk = base + b * IB
            pltpu.sync_copy(src_hbm.at[pl.ds(blk, IB)], src_v)
            pltpu.sync_copy(dst_hbm.at[pl.ds(blk, IB)], dst_v)
            dg = [None] * IB
            dg[0] = pltpu.async_copy(feat_hbm.at[src_v.at[0]], rows[0], sem_g[0])
            for j in range(IB):
                p = j % 2
                if j + 1 < IB:
                    dg[j + 1] = pltpu.async_copy(
                        feat_hbm.at[src_v.at[j + 1]], rows[1 - p], sem_g[1 - p])
                dg[j].wait()
                pltpu.sync_copy(rows[p], acc_sh.at[dst_v.at[j]], add=True)
                if with_cnt:
                    @pl.loop(0, CHUNK // 16)
                    def _(k):
                        idx = dst_v[j, pl.ds(k * 16, 16)]
                        plsc.addupdate_scatter(hist_v, [idx], ones16)

        plsc.subcore_barrier()

        pltpu.sync_copy(acc_sh.at[pl.ds(row0, RPT)], sum_hbm.at[c, pl.ds(row0, RPT)])
        if with_cnt:
            pltpu.sync_copy(hist_v, cnt_hbm.at[wid])

    cp = pltpu.CompilerParams()
    if "needs_layout_passes" in pltpu.CompilerParams.__dataclass_fields__:
        cp = dataclasses.replace(cp, needs_layout_passes=False)
    return pl.kernel(body, out_type=tuple(out_type) if with_cnt else out_type[0],
                     mesh=mesh, scratch_types=scratch, compiler_params=cp)


BLK = 2048


def _tc_layer(sum_parts, inv_cnt, feats, Wl, bl, Wr, relu):
    def body(sum_ref, inv_ref, x_ref, wl_ref, bl_ref, wr_ref, o_ref):
        mean = (sum_ref[0] + sum_ref[1]) * inv_ref[...]
        acc = lax.dot_general(mean, wl_ref[...], (((1,), (1,)), ((), ())),
                              preferred_element_type=jnp.float32)
        acc = acc + lax.dot_general(x_ref[...], wr_ref[...],
                                    (((1,), (1,)), ((), ())),
                                    preferred_element_type=jnp.float32)
        acc = acc + bl_ref[...]
        if relu:
            acc = jnp.maximum(acc, 0.0)
        o_ref[...] = acc

    return pl.pallas_call(
        body,
        grid=(NP // BLK,),
        in_specs=[
            pl.BlockSpec((2, BLK, D), lambda i: (0, i, 0)),
            pl.BlockSpec((BLK, 1), lambda i: (i, 0)),
            pl.BlockSpec((BLK, D), lambda i: (i, 0)),
            pl.BlockSpec((D, D), lambda i: (0, 0)),
            pl.BlockSpec((1, D), lambda i: (0, 0)),
            pl.BlockSpec((D, D), lambda i: (0, 0)),
        ],
        out_specs=pl.BlockSpec((BLK, D), lambda i: (i, 0)),
        out_shape=jax.ShapeDtypeStruct((NP, D), jnp.float32),
    )(sum_parts, inv_cnt, feats, Wl, bl.reshape(1, D), Wr)


def kernel(x, edge_index, W1l, b1, W1r, W2l, b2, W2r):
    pad = EP - E
    src_flat = jnp.concatenate(
        [edge_index[0].astype(jnp.int32), jnp.zeros((pad,), jnp.int32)]
    )
    dst_flat = jnp.concatenate(
        [edge_index[1].astype(jnp.int32),
         N + (jnp.arange(pad, dtype=jnp.int32) % (NP - N))]
    )
    x_pad = jnp.pad(x, ((0, NP - N), (0, 0)))
    sum1, cnt_parts = _sc_aggregate(True)(
        x_pad, src_flat.reshape(-1, 64), dst_flat.reshape(-1, 64))
    inv_cnt = (1.0 / jnp.maximum(cnt_parts.sum(axis=0), 1.0)).reshape(NP, 1)
    h = _tc_layer(sum1, inv_cnt, x_pad, W1l, b1, W1r, relu=True)
    sum2 = _sc_aggregate(False)(
        h, src_flat.reshape(-1, 64), dst_flat.reshape(-1, 64))
    out = _tc_layer(sum2, inv_cnt, h, W2l, b2, W2r, relu=False)
    return out[:N]
